# Initial kernel scaffold; baseline (speedup 1.0000x reference)
import jax
import jax.numpy as jnp
from jax.experimental import pallas as pl


def kernel(hidden_states, gate_weight, e_score_correction_bias, gate_proj, up_proj, down_proj, shared_gate, shared_up, shared_down):
    raise NotImplementedError("write your pallas kernel here")


# speedup vs baseline: 2.5833x; 2.5833x over previous
import functools

import jax
import jax.numpy as jnp
from jax.experimental import pallas as pl
from jax.experimental.pallas import tpu as pltpu

E = 64
TOP_K = 2
D_MODEL = 1024
D_FF = 512
SEQ = 2048
ROUTED_SCALING = 2.5


def _router_body(x_ref, gw_ref, bias_ref, pew_ref):
    x = x_ref[...]
    logits = jax.lax.dot_general(x, gw_ref[...], (((1,), (1,)), ((), ())),
                                 preferred_element_type=jnp.float32)
    scores = jax.nn.sigmoid(logits)
    choice = scores + bias_ref[...]
    lane = jax.lax.broadcasted_iota(jnp.int32, (SEQ, E), 1).astype(jnp.float32)
    m1 = jnp.max(choice, axis=1, keepdims=True)
    idx1 = jnp.min(jnp.where(choice == m1, lane, float(E)), axis=1, keepdims=True)
    is1 = lane == idx1
    w1 = jnp.sum(jnp.where(is1, scores, 0.0), axis=1, keepdims=True)
    choice2 = jnp.where(is1, -jnp.inf, choice)
    m2 = jnp.max(choice2, axis=1, keepdims=True)
    idx2 = jnp.min(jnp.where(choice2 == m2, lane, float(E)), axis=1, keepdims=True)
    is2 = lane == idx2
    w2 = jnp.sum(jnp.where(is2, scores, 0.0), axis=1, keepdims=True)
    denom = w1 + w2 + 1e-20
    w1n = w1 / denom * ROUTED_SCALING
    w2n = w2 / denom * ROUTED_SCALING
    pew_ref[...] = jnp.where(is1, w1n, 0.0) + jnp.where(is2, w2n, 0.0)


def _moe_body(x_ref, pew_ref, gp_ref, up_ref, dp_ref, sg_ref, su_ref, sd_ref, out_ref):
    e = pl.program_id(0)
    x = x_ref[...]

    @pl.when(e == 0)
    def _init():
        h = jax.nn.silu(
            jax.lax.dot_general(x, sg_ref[...], (((1,), (1,)), ((), ())),
                                preferred_element_type=jnp.float32)
        ) * jax.lax.dot_general(x, su_ref[...], (((1,), (1,)), ((), ())),
                                preferred_element_type=jnp.float32)
        out_ref[...] = jax.lax.dot_general(h, sd_ref[...], (((1,), (1,)), ((), ())),
                                           preferred_element_type=jnp.float32)

    gw = gp_ref[0]
    uw = up_ref[0]
    dw = dp_ref[0]
    h = jax.nn.silu(
        jax.lax.dot_general(x, gw, (((1,), (1,)), ((), ())),
                            preferred_element_type=jnp.float32)
    ) * jax.lax.dot_general(x, uw, (((1,), (1,)), ((), ())),
                            preferred_element_type=jnp.float32)
    y = jax.lax.dot_general(h, dw, (((1,), (1,)), ((), ())),
                            preferred_element_type=jnp.float32)
    sel = (jax.lax.broadcasted_iota(jnp.int32, (E, 1), 0) == e).astype(jnp.float32)
    w = jax.lax.dot_general(pew_ref[...], sel, (((1,), (0,)), ((), ())),
                            preferred_element_type=jnp.float32)
    out_ref[...] += y * w


def kernel(hidden_states, gate_weight, e_score_correction_bias, gate_proj,
           up_proj, down_proj, shared_gate, shared_up, shared_down):
    b, s, d = hidden_states.shape
    x = hidden_states.reshape(s, d)

    pew = pl.pallas_call(
        _router_body,
        out_shape=jax.ShapeDtypeStruct((SEQ, E), jnp.float32),
    )(x, gate_weight, e_score_correction_bias.reshape(1, E))

    out = pl.pallas_call(
        _moe_body,
        grid=(E,),
        in_specs=[
            pl.BlockSpec((SEQ, D_MODEL), lambda e: (0, 0)),
            pl.BlockSpec((SEQ, E), lambda e: (0, 0)),
            pl.BlockSpec((1, D_FF, D_MODEL), lambda e: (e, 0, 0)),
            pl.BlockSpec((1, D_FF, D_MODEL), lambda e: (e, 0, 0)),
            pl.BlockSpec((1, D_MODEL, D_FF), lambda e: (e, 0, 0)),
            pl.BlockSpec((D_FF, D_MODEL), lambda e: (0, 0)),
            pl.BlockSpec((D_FF, D_MODEL), lambda e: (0, 0)),
            pl.BlockSpec((D_MODEL, D_FF), lambda e: (0, 0)),
        ],
        out_specs=pl.BlockSpec((SEQ, D_MODEL), lambda e: (0, 0)),
        out_shape=jax.ShapeDtypeStruct((SEQ, D_MODEL), jnp.float32),
    )(x, pew, gate_proj, up_proj, down_proj, shared_gate, shared_up, shared_down)

    return out.reshape(b, s, d)
